# baseline (speedup 1.0000x reference)
import jax
import jax.numpy as jnp
from jax import lax
from jax.experimental import pallas as pl
from jax.experimental.pallas import tpu as pltpu
from jax.experimental.pallas import tpu_sc as plsc

NC = 2
NS = 16
L = 16
NW = NC * NS

CHUNK = 128


def _pairmf_body(u_hbm, i_hbm, j_hbm, c_hbm, emb_hbm,
                 pi_hbm, pj_hbm,
                 idx_v, rows_u, rows_i, rows_j, rows_c,
                 pi_v, pj_v, sem):
    b_per_w = rows_u.shape[0]
    n_chunks = b_per_w // CHUNK
    D = rows_u.shape[1]
    wid = lax.axis_index("s") * NC + lax.axis_index("c")
    base = wid * b_per_w

    idx_copies = []
    for a, src in enumerate((u_hbm, i_hbm, j_hbm, c_hbm)):
        for k in range(n_chunks):
            idx_copies.append(pltpu.async_copy(
                src.at[pl.ds(base + k * CHUNK, CHUNK)], idx_v.at[a, k], sem))
    for cp in idx_copies:
        cp.wait()

    row_copies = []
    for a, dst in enumerate((rows_u, rows_i, rows_j, rows_c)):
        for k in range(n_chunks):
            row_copies.append(pltpu.async_copy(
                emb_hbm.at[idx_v.at[a, k]],
                dst.at[pl.ds(k * CHUNK, CHUNK)], sem))
    for cp in row_copies:
        cp.wait()

    lanes = lax.iota(jnp.int32, L)

    @pl.loop(0, b_per_w // L)
    def _block(t):
        row = t * L + lanes
        acc_i = jnp.zeros((L,), jnp.float32)
        acc_j = jnp.zeros((L,), jnp.float32)
        for d in range(D):
            col = jnp.full((L,), d, jnp.int32)
            eu = plsc.load_gather(rows_u, [row, col])
            ec = plsc.load_gather(rows_c, [row, col])
            ei = plsc.load_gather(rows_i, [row, col])
            ej = plsc.load_gather(rows_j, [row, col])
            t3 = eu * ec
            acc_i = acc_i + t3 * ei
            acc_j = acc_j + t3 * ej
        pi_v[pl.ds(t * L, L)] = acc_i
        pj_v[pl.ds(t * L, L)] = acc_j

    pltpu.sync_copy(pi_v, pi_hbm.at[pl.ds(base, b_per_w)])
    pltpu.sync_copy(pj_v, pj_hbm.at[pl.ds(base, b_per_w)])


def kernel(u, i, j, context, emb_weight):
    B = u.shape[0]
    V, D = emb_weight.shape
    assert B % (NW * L) == 0 and (B // NW) % CHUNK == 0
    b_per_w = B // NW
    n_chunks = b_per_w // CHUNK

    mesh = plsc.VectorSubcoreMesh(core_axis_name="c", subcore_axis_name="s",
                                  num_cores=NC, num_subcores=NS)
    run = pl.kernel(
        _pairmf_body,
        out_type=(jax.ShapeDtypeStruct((B,), jnp.float32),
                  jax.ShapeDtypeStruct((B,), jnp.float32)),
        mesh=mesh,
        compiler_params=pltpu.CompilerParams(needs_layout_passes=False,
                                             use_tc_tiling_on_sc=False),
        scratch_types=[
            pltpu.VMEM((4, n_chunks, CHUNK), jnp.int32),
            pltpu.VMEM((b_per_w, D), jnp.float32),
            pltpu.VMEM((b_per_w, D), jnp.float32),
            pltpu.VMEM((b_per_w, D), jnp.float32),
            pltpu.VMEM((b_per_w, D), jnp.float32),
            pltpu.VMEM((b_per_w,), jnp.float32),
            pltpu.VMEM((b_per_w,), jnp.float32),
            pltpu.SemaphoreType.DMA,
        ],
    )
    return run(u.astype(jnp.int32), i.astype(jnp.int32),
               j.astype(jnp.int32), context.astype(jnp.int32),
               emb_weight)
